# Initial kernel scaffold; baseline (speedup 1.0000x reference)
import jax
import jax.numpy as jnp
from jax.experimental import pallas as pl


def kernel(x, W_proj, b_proj, W_out, b_out):
    raise NotImplementedError("write your pallas kernel here")


# speedup vs baseline: 18.7923x; 18.7923x over previous
import functools

import jax
import jax.numpy as jnp
from jax.experimental import pallas as pl

B, T, C = 8, 1024, 256
H = 128
K = 32
BIG = 1e9
INF = 3.0e38


def _encoder_kernel(x_ref, wp_ref, bp_ref, wo_ref, bo_ref, out_ref):
    x = x_ref[0]
    wp = wp_ref[...]
    h = jax.lax.dot_general(
        x, wp, (((1,), (1,)), ((), ())),
        preferred_element_type=jnp.float32,
        precision=jax.lax.Precision.HIGHEST,
    ) + bp_ref[...]

    sq = jnp.sum(h * h, axis=1, keepdims=True)
    g = jax.lax.dot_general(
        h, h, (((1,), (1,)), ((), ())),
        preferred_element_type=jnp.float32,
        precision=jax.lax.Precision.HIGHEST,
    )
    d2 = sq + jnp.transpose(sq) - 2.0 * g
    d2 = jnp.maximum(d2, 0.0)
    row = jax.lax.broadcasted_iota(jnp.int32, (T, T), 0)
    col = jax.lax.broadcasted_iota(jnp.int32, (T, T), 1)
    d2 = jnp.where(row == col, BIG, d2)

    def step(_, t):
        masked = jnp.where(d2 <= t, INF, d2)
        return jnp.min(masked, axis=1, keepdims=True)
    thr = jax.lax.fori_loop(0, K, step, jnp.full((T, 1), -1.0, jnp.float32))

    w = jnp.where(d2 <= thr, jnp.exp(d2 * (-1.0 / (2.0 + 1e-8))), 0.0)
    z = jnp.sum(w, axis=1, keepdims=True) + 1e-8
    wn = w / z
    smooth = jax.lax.dot_general(
        wn, h, (((1,), (0,)), ((), ())),
        preferred_element_type=jnp.float32,
        precision=jax.lax.Precision.HIGHEST,
    )
    lap = jnp.tanh(h - smooth)
    out = jax.lax.dot_general(
        lap, wo_ref[...], (((1,), (1,)), ((), ())),
        preferred_element_type=jnp.float32,
        precision=jax.lax.Precision.HIGHEST,
    ) + bo_ref[...]
    out_ref[0] = out


@jax.jit
def kernel(x, W_proj, b_proj, W_out, b_out):
    return pl.pallas_call(
        _encoder_kernel,
        grid=(B,),
        in_specs=[
            pl.BlockSpec((1, T, C), lambda b: (b, 0, 0)),
            pl.BlockSpec((H, C), lambda b: (0, 0)),
            pl.BlockSpec((1, H), lambda b: (0, 0)),
            pl.BlockSpec((H, H), lambda b: (0, 0)),
            pl.BlockSpec((1, H), lambda b: (0, 0)),
        ],
        out_specs=pl.BlockSpec((1, T, H), lambda b: (b, 0, 0)),
        out_shape=jax.ShapeDtypeStruct((B, T, H), jnp.float32),
    )(x, W_proj, b_proj.reshape(1, H), W_out, b_out.reshape(1, H))
